# baseline (speedup 1.0000x reference)
import functools

import jax
import jax.numpy as jnp
from jax import lax
from jax.experimental import pallas as pl
from jax.experimental.pallas import tpu as pltpu
from jax.experimental.pallas import tpu_sc as plsc


def _emb_body(ids_hbm, word_hbm, pos_hbm, out_hbm,
              idx_v, poff_v, pos_v, rows_v, obuf_v,
              gsem0, gsem1, ssem0, ssem1,
              *, cpw, chunk, seq, d, nc, rpw):
    wid = lax.axis_index("s") * nc + lax.axis_index("c")
    gsem = (gsem0, gsem1)
    ssem = (ssem0, ssem1)
    base = wid * rpw

    pltpu.sync_copy(ids_hbm.at[wid], idx_v)
    pltpu.sync_copy(pos_hbm.at[pl.ds(0, seq * d)], pos_v)

    def body_pre(j, _):
        sl = pl.ds(j * 16, 16)
        k = base + j * 16 + lax.iota(jnp.int32, 16)
        poff_v[sl] = lax.shift_left(lax.rem(k, seq), 6)
        return 0

    lax.fori_loop(0, rpw // 16, body_pre, 0, unroll=4)

    def gather_start(b, c):
        pltpu.async_copy(word_hbm.at[idx_v.at[pl.ds(c * chunk, chunk)]],
                         rows_v.at[b], gsem[b])

    def gather_wait(b, c):
        pltpu.make_async_copy(word_hbm.at[idx_v.at[pl.ds(c * chunk, chunk)]],
                              rows_v.at[b], gsem[b]).wait()

    def out_slice(c):
        return out_hbm.at[pl.ds((wid * cpw + c) * (chunk // 2), chunk // 2)]

    def store_start(b, c):
        pltpu.async_copy(obuf_v.at[b], out_slice(c), ssem[b])

    def store_wait(b, c):
        pltpu.make_async_copy(obuf_v.at[b], out_slice(c), ssem[b]).wait()

    def extract(b, c):
        def body_q(q, _):
            r0 = c * chunk + q * 16
            pv = poff_v[pl.ds(r0, 16)]
            for k2 in range(16):
                k = q * 16 + k2
                p = pv[k2]
                kr = q * 8 + k2 // 2
                ko = (k2 % 2) * 64
                for t in range(4):
                    obuf_v[b, kr, pl.ds(ko + t * 16, 16)] = (
                        rows_v[b, k, pl.ds(t * 16, 16)]
                        + pos_v[pl.ds(p + t * 16, 16)])
            return 0

        lax.fori_loop(0, chunk // 16, body_q, 0)

    gather_start(0, 0)

    def body_pair(g, _):
        for b in range(2):
            c = g * 2 + b
            nb = 1 - b
            @pl.when(c >= 1)
            def _():
                store_wait(nb, c - 1)

            @pl.when(c + 1 < cpw)
            def _():
                gather_start(nb, c + 1)

            gather_wait(b, c)
            extract(b, c)
            store_start(b, c)
        return 0

    lax.fori_loop(0, cpw // 2, body_pair, 0)
    store_wait(1, cpw - 1)


def _transpose_block(x_ref, o_ref, *, bcol, d):
    x = x_ref[...]
    xt = jnp.transpose(x, (1, 0))
    xt3 = jnp.reshape(xt, (bcol // 2, 2, d))
    o_ref[...] = jnp.concatenate([xt3[:, 0, :], xt3[:, 1, :]], axis=1)


def _retile_table(word_t):
    d, v = word_t.shape
    bcol = 2048
    grid = (v + bcol - 1) // bcol
    return pl.pallas_call(
        functools.partial(_transpose_block, bcol=bcol, d=d),
        grid=(grid,),
        in_specs=[pl.BlockSpec((d, bcol), lambda i: (0, i))],
        out_specs=pl.BlockSpec((bcol // 2, 2 * d), lambda i: (i, 0)),
        out_shape=jax.ShapeDtypeStruct((v // 2, 2 * d), jnp.float32),
    )(word_t)


def _transpose_out_block(x_ref, o_ref):
    o_ref[...] = jnp.transpose(x_ref[...], (1, 0))


def _transpose_out(x, b, sd):
    bb, bs = 256, 1280
    return pl.pallas_call(
        _transpose_out_block,
        grid=(b // bb, sd // bs),
        in_specs=[pl.BlockSpec((bb, bs), lambda i, j: (i, j))],
        out_specs=pl.BlockSpec((bs, bb), lambda i, j: (j, i)),
        out_shape=jax.ShapeDtypeStruct((sd, b), jnp.float32),
    )(x)


def kernel(input_ids, word_emb, pos_emb):
    b, s = input_ids.shape
    v, d = word_emb.shape
    n = b * s

    nc, ns = 2, 16
    nw = nc * ns
    chunk = 128
    rpw = n // nw
    cpw = rpw // chunk

    ids = input_ids.reshape(nw, rpw).astype(jnp.int32)
    word2 = _retile_table(word_emb.T).reshape(v, d)
    pos_flat = pos_emb[:s].reshape(s * d)

    mesh = plsc.VectorSubcoreMesh(core_axis_name="c", subcore_axis_name="s")
    fn = pl.kernel(
        functools.partial(_emb_body, cpw=cpw, chunk=chunk, seq=s, d=d,
                          nc=nc, rpw=rpw),
        mesh=mesh,
        compiler_params=pltpu.CompilerParams(use_tc_tiling_on_sc=False),
        out_type=jax.ShapeDtypeStruct((n * d // (2 * d), 2 * d), jnp.float32),
        scratch_types=[
            pltpu.VMEM((rpw,), jnp.int32),
            pltpu.VMEM((rpw,), jnp.int32),
            pltpu.VMEM((s * d,), jnp.float32),
            pltpu.VMEM((2, chunk, d), jnp.float32),
            pltpu.VMEM((2, chunk // 2, 2 * d), jnp.float32),
            pltpu.SemaphoreType.DMA,
            pltpu.SemaphoreType.DMA,
            pltpu.SemaphoreType.DMA,
            pltpu.SemaphoreType.DMA,
        ],
    )
    out = fn(ids, word2, pos_flat)
    out_t = _transpose_out(out.reshape(b, s * d), b, s * d)
    return out_t.reshape(s, d, b).transpose(2, 0, 1)


# speedup vs baseline: 1.2059x; 1.2059x over previous
import functools

import jax
import jax.numpy as jnp
from jax import lax
from jax.experimental import pallas as pl
from jax.experimental.pallas import tpu as pltpu
from jax.experimental.pallas import tpu_sc as plsc


def _emb_body(ids_hbm, word_hbm, pos_hbm, out_hbm,
              idx_v, poff_v, pos_v, rows_v, obuf_v,
              gsem0, gsem1, ssem0, ssem1,
              *, cpw, chunk, seq, d, nc, rpw):
    wid = lax.axis_index("s") * nc + lax.axis_index("c")
    gsem = (gsem0, gsem1)
    ssem = (ssem0, ssem1)
    base = wid * rpw

    pltpu.sync_copy(ids_hbm.at[wid], idx_v)
    pltpu.sync_copy(pos_hbm.at[pl.ds(0, seq * d)], pos_v)

    def body_pre(j, _):
        sl = pl.ds(j * 16, 16)
        k = base + j * 16 + lax.iota(jnp.int32, 16)
        poff_v[sl] = lax.shift_left(lax.rem(k, seq), 6)
        return 0

    lax.fori_loop(0, rpw // 16, body_pre, 0, unroll=4)

    def gather_start(b, c):
        pltpu.async_copy(word_hbm.at[idx_v.at[pl.ds(c * chunk, chunk)]],
                         rows_v.at[b], gsem[b])

    def gather_wait(b, c):
        pltpu.make_async_copy(word_hbm.at[idx_v.at[pl.ds(c * chunk, chunk)]],
                              rows_v.at[b], gsem[b]).wait()

    def out_slice(c):
        return out_hbm.at[pl.ds((wid * cpw + c) * (chunk // 2), chunk // 2)]

    def store_start(b, c):
        pltpu.async_copy(obuf_v.at[b], out_slice(c), ssem[b])

    def store_wait(b, c):
        pltpu.make_async_copy(obuf_v.at[b], out_slice(c), ssem[b]).wait()

    def extract(b, c):
        def body_q(q, _):
            r0 = c * chunk + q * 16
            pv = poff_v[pl.ds(r0, 16)]
            for k2 in range(16):
                k = q * 16 + k2
                p = pv[k2]
                kr = q * 8 + k2 // 2
                ko = (k2 % 2) * 64
                for t in range(4):
                    obuf_v[b, kr, pl.ds(ko + t * 16, 16)] = (
                        rows_v[b, k, pl.ds(t * 16, 16)]
                        + pos_v[pl.ds(p + t * 16, 16)])
            return 0

        lax.fori_loop(0, chunk // 16, body_q, 0)

    gather_start(0, 0)

    def body_pair(g, _):
        for b in range(2):
            c = g * 2 + b
            nb = 1 - b
            @pl.when(c >= 1)
            def _():
                store_wait(nb, c - 1)

            @pl.when(c + 1 < cpw)
            def _():
                gather_start(nb, c + 1)

            gather_wait(b, c)
            extract(b, c)
            store_start(b, c)
        return 0

    lax.fori_loop(0, cpw // 2, body_pair, 0)
    store_wait(1, cpw - 1)


def _transpose_block(x_ref, o_ref, *, bcol, d):
    x = x_ref[...]
    xt = jnp.transpose(x, (1, 0))
    xt3 = jnp.reshape(xt, (bcol // 2, 2, d))
    o_ref[...] = jnp.concatenate([xt3[:, 0, :], xt3[:, 1, :]], axis=1)


def _retile_table(word_t):
    d, v = word_t.shape
    bcol = 16384
    grid = (v + bcol - 1) // bcol
    return pl.pallas_call(
        functools.partial(_transpose_block, bcol=bcol, d=d),
        grid=(grid,),
        in_specs=[pl.BlockSpec((d, bcol), lambda i: (0, i))],
        out_specs=pl.BlockSpec((bcol // 2, 2 * d), lambda i: (i, 0)),
        out_shape=jax.ShapeDtypeStruct((v // 2, 2 * d), jnp.float32),
    )(word_t)


def _transpose_out_block(x_ref, o_ref):
    o_ref[...] = jnp.transpose(x_ref[...], (1, 0))


def _transpose_out(x, b, sd):
    bb, bs = 256, 1280
    return pl.pallas_call(
        _transpose_out_block,
        grid=(b // bb, sd // bs),
        in_specs=[pl.BlockSpec((bb, bs), lambda i, j: (i, j))],
        out_specs=pl.BlockSpec((bs, bb), lambda i, j: (j, i)),
        out_shape=jax.ShapeDtypeStruct((sd, b), jnp.float32),
    )(x)


def kernel(input_ids, word_emb, pos_emb):
    b, s = input_ids.shape
    v, d = word_emb.shape
    n = b * s

    nc, ns = 2, 16
    nw = nc * ns
    chunk = 128
    rpw = n // nw
    cpw = rpw // chunk

    ids = input_ids.reshape(nw, rpw).astype(jnp.int32)
    word2 = _retile_table(word_emb.T).reshape(v, d)
    pos_flat = pos_emb[:s].reshape(s * d)

    mesh = plsc.VectorSubcoreMesh(core_axis_name="c", subcore_axis_name="s")
    fn = pl.kernel(
        functools.partial(_emb_body, cpw=cpw, chunk=chunk, seq=s, d=d,
                          nc=nc, rpw=rpw),
        mesh=mesh,
        compiler_params=pltpu.CompilerParams(use_tc_tiling_on_sc=False),
        out_type=jax.ShapeDtypeStruct((n * d // (2 * d), 2 * d), jnp.float32),
        scratch_types=[
            pltpu.VMEM((rpw,), jnp.int32),
            pltpu.VMEM((rpw,), jnp.int32),
            pltpu.VMEM((s * d,), jnp.float32),
            pltpu.VMEM((2, chunk, d), jnp.float32),
            pltpu.VMEM((2, chunk // 2, 2 * d), jnp.float32),
            pltpu.SemaphoreType.DMA,
            pltpu.SemaphoreType.DMA,
            pltpu.SemaphoreType.DMA,
            pltpu.SemaphoreType.DMA,
        ],
    )
    out = fn(ids, word2, pos_flat)
    out_t = _transpose_out(out.reshape(b, s * d), b, s * d)
    return out_t.reshape(s, d, b).transpose(2, 0, 1)
